# baseline (speedup 1.0000x reference)
import functools

import jax
import jax.numpy as jnp
from jax import lax
from jax.experimental import pallas as pl
from jax.experimental.pallas import tpu as pltpu
from jax.experimental.pallas import tpu_sc as plsc

_TOPK = 10
_LANES = 16
_NTILES = 32


def _sc_thresholds(aff):
    B, P, N = aff.shape
    chunks_per_batch = N // _LANES
    n_chunks = B * chunks_per_batch
    per_tile = pl.cdiv(n_chunks, _NTILES)

    mesh = plsc.VectorSubcoreMesh(core_axis_name="c", subcore_axis_name="s")

    @functools.partial(
        pl.kernel,
        mesh=mesh,
        out_type=(
            jax.ShapeDtypeStruct((B, N), jnp.float32),
            jax.ShapeDtypeStruct((B, N), jnp.float32),
        ),
        scratch_types=[
            pltpu.VMEM((P, _LANES), jnp.float32),
            pltpu.VMEM((_LANES,), jnp.float32),
            pltpu.VMEM((_LANES,), jnp.float32),
        ],
        compiler_params=pltpu.CompilerParams(use_tc_tiling_on_sc=False),
    )
    def sc_kernel(aff_hbm, th_hbm, mx_hbm, buf, thv, mxv):
        wid = lax.axis_index("s") * 2 + lax.axis_index("c")

        def chunk_body(ci, carry):
            c = wid + ci * _NTILES

            @pl.when(c < n_chunks)
            def _():
                b = c // chunks_per_batch
                co = (c % chunks_per_batch) * _LANES
                pltpu.sync_copy(aff_hbm.at[b, :, pl.ds(co, _LANES)], buf)

                def row_body(i, t):
                    v = buf[i]
                    t = list(t)
                    for j in range(_TOPK):
                        hi = jnp.maximum(t[j], v)
                        v = jnp.minimum(t[j], v)
                        t[j] = hi
                    return tuple(t)

                neg_inf = jnp.full((_LANES,), -jnp.inf, dtype=jnp.float32)
                t = lax.fori_loop(
                    0, P, row_body, (neg_inf,) * _TOPK, unroll=8
                )
                thv[...] = t[_TOPK - 1]
                mxv[...] = t[0]
                pltpu.sync_copy(thv, th_hbm.at[b, pl.ds(co, _LANES)])
                pltpu.sync_copy(mxv, mx_hbm.at[b, pl.ds(co, _LANES)])

            return carry

        lax.fori_loop(0, per_tile, chunk_body, 0)

    return sc_kernel(aff)


def _tc_assemble_body(aff_ref, feat_ref, th_ref, mx_ref, out_ref):
    a = aff_ref[0]
    th = th_ref[0]
    mx = mx_ref[0]
    e = jnp.where(a >= th, jnp.exp(a - mx), 0.0)
    s = jnp.sum(e, axis=0, keepdims=True)
    f = feat_ref[0]
    acc = lax.dot_general(
        f, e, (((1,), (0,)), ((), ())), preferred_element_type=jnp.float32
    )
    out_ref[0] = acc * (1.0 / s)


def _tc_assemble(aff, feat, th, mx):
    B, P, N = aff.shape
    C = feat.shape[1]
    NB = 512
    grid = (B, pl.cdiv(N, NB))
    return pl.pallas_call(
        _tc_assemble_body,
        grid=grid,
        in_specs=[
            pl.BlockSpec((1, P, NB), lambda b, n: (b, 0, n)),
            pl.BlockSpec((1, C, P), lambda b, n: (b, 0, 0)),
            pl.BlockSpec((1, 1, NB), lambda b, n: (b, 0, n)),
            pl.BlockSpec((1, 1, NB), lambda b, n: (b, 0, n)),
        ],
        out_specs=pl.BlockSpec((1, C, NB), lambda b, n: (b, 0, n)),
        out_shape=jax.ShapeDtypeStruct((B, C, N), jnp.float32),
    )(aff, feat, th, mx)


def kernel(cur_prev_aff, feat):
    B, P, N = cur_prev_aff.shape
    th, mx = _sc_thresholds(cur_prev_aff)
    return _tc_assemble(
        cur_prev_aff, feat, th.reshape(B, 1, N), mx.reshape(B, 1, N)
    )


# speedup vs baseline: 1.3845x; 1.3845x over previous
import functools

import jax
import jax.numpy as jnp
from jax import lax
from jax.experimental import pallas as pl
from jax.experimental.pallas import tpu as pltpu
from jax.experimental.pallas import tpu_sc as plsc

_TOPK = 10
_LANES = 16
_NTILES = 32


def _sc_thresholds(aff):
    B, P, N = aff.shape
    chunks_per_batch = N // _LANES
    n_chunks = B * chunks_per_batch
    per_tile = pl.cdiv(n_chunks, _NTILES)

    mesh = plsc.VectorSubcoreMesh(core_axis_name="c", subcore_axis_name="s")

    @functools.partial(
        pl.kernel,
        mesh=mesh,
        out_type=(
            jax.ShapeDtypeStruct((B, N), jnp.float32),
            jax.ShapeDtypeStruct((B, N), jnp.float32),
        ),
        scratch_types=[
            pltpu.VMEM((P, _LANES), jnp.float32),
            pltpu.VMEM((_LANES,), jnp.float32),
            pltpu.VMEM((_LANES,), jnp.float32),
        ],
        compiler_params=pltpu.CompilerParams(use_tc_tiling_on_sc=False),
    )
    def sc_kernel(aff_hbm, th_hbm, mx_hbm, buf, thv, mxv):
        wid = lax.axis_index("s") * 2 + lax.axis_index("c")

        def chunk_body(ci, carry):
            c = wid + ci * _NTILES

            @pl.when(c < n_chunks)
            def _():
                b = c // chunks_per_batch
                co = (c % chunks_per_batch) * _LANES
                pltpu.sync_copy(aff_hbm.at[b, :, pl.ds(co, _LANES)], buf)

                def row_body(i, t):
                    v = buf[i]
                    t = list(t)
                    for j in range(_TOPK):
                        hi = jnp.maximum(t[j], v)
                        v = jnp.minimum(t[j], v)
                        t[j] = hi
                    return tuple(t)

                neg_inf = jnp.full((_LANES,), -jnp.inf, dtype=jnp.float32)
                t = lax.fori_loop(
                    0, P, row_body, (neg_inf,) * _TOPK, unroll=8
                )
                thv[...] = t[_TOPK - 1]
                mxv[...] = t[0]
                pltpu.sync_copy(thv, th_hbm.at[b, pl.ds(co, _LANES)])
                pltpu.sync_copy(mxv, mx_hbm.at[b, pl.ds(co, _LANES)])

            return carry

        lax.fori_loop(0, per_tile, chunk_body, 0)

    return sc_kernel(aff)


def _tc_assemble_body(aff_ref, feat_ref, th_ref, mx_ref, out_ref):
    a = aff_ref[0]
    th = th_ref[0]
    mx = mx_ref[0]
    e = jnp.where(a >= th, jnp.exp(a - mx), 0.0)
    s = jnp.sum(e, axis=0, keepdims=True)
    f = feat_ref[0]
    acc = lax.dot_general(
        f, e, (((1,), (0,)), ((), ())), preferred_element_type=jnp.float32
    )
    out_ref[0] = acc * (1.0 / s)


def _tc_assemble(aff, feat, th, mx):
    B, P, N = aff.shape
    C = feat.shape[1]
    NB = 512
    grid = (B, pl.cdiv(N, NB))
    return pl.pallas_call(
        _tc_assemble_body,
        grid=grid,
        in_specs=[
            pl.BlockSpec((1, P, NB), lambda b, n: (b, 0, n)),
            pl.BlockSpec((1, C, P), lambda b, n: (b, 0, 0)),
            pl.BlockSpec((1, 1, NB), lambda b, n: (b, 0, n)),
            pl.BlockSpec((1, 1, NB), lambda b, n: (b, 0, n)),
        ],
        out_specs=pl.BlockSpec((1, C, NB), lambda b, n: (b, 0, n)),
        out_shape=jax.ShapeDtypeStruct((B, C, N), jnp.float32),
    )(aff, feat, th, mx)


def kernel(cur_prev_aff, feat):
    B, P, N = cur_prev_aff.shape
    a0 = cur_prev_aff[:1]
    a1 = cur_prev_aff[1:]
    th1, mx1 = _sc_thresholds(a1)
    out0 = _tc_full(a0, feat[:1])
    out1 = _tc_assemble(a1, feat[1:], th1.reshape(1, 1, N), mx1.reshape(1, 1, N))
    return jnp.concatenate([out0, out1], axis=0)


def _tc_full_body(aff_ref, feat_ref, out_ref):
    p = aff_ref.shape[1]
    nb = aff_ref.shape[2]
    s = 8

    def _insert(i, t):
        v = aff_ref[0, pl.ds(i * s, s), :]
        t = list(t)
        for j in range(_TOPK):
            hi = jnp.maximum(t[j], v)
            v = jnp.minimum(t[j], v)
            t[j] = hi
        return tuple(t)

    t0 = tuple(
        jnp.full((s, nb), -jnp.inf, dtype=jnp.float32) for _ in range(_TOPK)
    )
    t = lax.fori_loop(0, p // s, _insert, t0, unroll=8)

    cand = jnp.concatenate(list(t), axis=0)
    vals = cand
    need = jnp.full((1, nb), float(_TOPK), dtype=jnp.float32)
    th = jnp.full((1, nb), -jnp.inf, dtype=jnp.float32)
    for _ in range(_TOPK):
        m = jnp.max(vals, axis=0, keepdims=True)
        ge = vals >= m
        c = jnp.sum(ge.astype(jnp.float32), axis=0, keepdims=True)
        th = jnp.where(need > 0.0, m, th)
        need = need - c
        vals = jnp.where(ge, -jnp.inf, vals)

    a = aff_ref[0]
    mx = jnp.max(t[0], axis=0, keepdims=True)
    e = jnp.where(a >= th, jnp.exp(a - mx), 0.0)
    s_ = jnp.sum(e, axis=0, keepdims=True)
    f = feat_ref[0]
    acc = lax.dot_general(
        f, e, (((1,), (0,)), ((), ())), preferred_element_type=jnp.float32
    )
    out_ref[0] = acc * (1.0 / s_)


def _tc_full(aff, feat):
    B, P, N = aff.shape
    C = feat.shape[1]
    NB = 512
    grid = (B, pl.cdiv(N, NB))
    return pl.pallas_call(
        _tc_full_body,
        grid=grid,
        in_specs=[
            pl.BlockSpec((1, P, NB), lambda b, n: (b, 0, n)),
            pl.BlockSpec((1, C, P), lambda b, n: (b, 0, 0)),
        ],
        out_specs=pl.BlockSpec((1, C, NB), lambda b, n: (b, 0, n)),
        out_shape=jax.ShapeDtypeStruct((B, C, N), jnp.float32),
    )(aff, feat)


# speedup vs baseline: 2.9063x; 2.0991x over previous
import jax
import jax.numpy as jnp
from jax.experimental import pallas as pl

_TOPK = 10


_ROWS_PER_CHUNK = 8
_N_STREAMS = 2
_UNROLL = 4
_NB = 256


def _assemble_body(aff_ref, feat_ref, out_ref):
    p = aff_ref.shape[1]
    nb = aff_ref.shape[2]
    s = _ROWS_PER_CHUNK

    ns = _N_STREAMS

    def _insert(i, t):
        t = list(t)
        for k in range(ns):
            v = aff_ref[0, pl.ds((i * ns + k) * s, s), :]
            for j in range(_TOPK):
                idx = k * _TOPK + j
                hi = jnp.maximum(t[idx], v)
                v = jnp.minimum(t[idx], v)
                t[idx] = hi
        return tuple(t)

    t0 = tuple(
        jnp.full((s, nb), -jnp.inf, dtype=jnp.float32)
        for _ in range(_TOPK * ns)
    )
    t = jax.lax.fori_loop(0, p // (s * ns), _insert, t0, unroll=_UNROLL)

    cand = jnp.concatenate(list(t), axis=0)
    vals = cand
    need = jnp.full((1, nb), float(_TOPK), dtype=jnp.float32)
    th = jnp.full((1, nb), -jnp.inf, dtype=jnp.float32)
    for _ in range(_TOPK):
        m = jnp.max(vals, axis=0, keepdims=True)
        ge = vals >= m
        c = jnp.sum(ge.astype(jnp.float32), axis=0, keepdims=True)
        th = jnp.where(need > 0.0, m, th)
        need = need - c
        vals = jnp.where(ge, -jnp.inf, vals)

    a = aff_ref[0]
    heads = jnp.concatenate([t[k * _TOPK] for k in range(ns)], axis=0)
    mx = jnp.max(heads, axis=0, keepdims=True)
    e = jnp.where(a >= th, jnp.exp(a - mx), 0.0)
    s = jnp.sum(e, axis=0, keepdims=True)

    f = feat_ref[0]
    acc = jax.lax.dot_general(
        f, e, (((1,), (0,)), ((), ())), preferred_element_type=jnp.float32
    )
    out_ref[0] = acc * (1.0 / s)


def kernel(cur_prev_aff, feat):
    B, P, N = cur_prev_aff.shape
    C = feat.shape[1]
    NB = _NB
    grid = (B, pl.cdiv(N, NB))
    return pl.pallas_call(
        _assemble_body,
        grid=grid,
        in_specs=[
            pl.BlockSpec((1, P, NB), lambda b, n: (b, 0, n)),
            pl.BlockSpec((1, C, P), lambda b, n: (b, 0, 0)),
        ],
        out_specs=pl.BlockSpec((1, C, NB), lambda b, n: (b, 0, n)),
        out_shape=jax.ShapeDtypeStruct((B, C, N), jnp.float32),
    )(cur_prev_aff, feat)


# speedup vs baseline: 3.7672x; 1.2962x over previous
import jax
import jax.numpy as jnp
from jax.experimental import pallas as pl
from jax.experimental.pallas import tpu as pltpu

_TOPK = 10
_S = 8
_G = 16
_KEEP = 3
_NB = 512


def _merge_sorted(a, b, keep):
    out = []
    for j in range(keep):
        terms = []
        if j < len(a):
            terms.append(a[j])
        if j < len(b):
            terms.append(b[j])
        for i in range(j):
            if i < len(a) and (j - 1 - i) < len(b):
                terms.append(jnp.minimum(a[i], b[j - 1 - i]))
        r = terms[0]
        for x in terms[1:]:
            r = jnp.maximum(r, x)
        out.append(r)
    return out


def _top3_of8(vs):
    pairs = []
    for i in range(4):
        hi = jnp.maximum(vs[2 * i], vs[2 * i + 1])
        lo = jnp.minimum(vs[2 * i], vs[2 * i + 1])
        pairs.append([hi, lo])
    m0 = _merge_sorted(pairs[0], pairs[1], 3)
    m1 = _merge_sorted(pairs[2], pairs[3], 3)
    return _merge_sorted(m0, m1, 3)


def _insertion_topk(read, nchunks, s, nb, unroll):
    def _insert(i, t):
        v = read(i)
        t = list(t)
        for j in range(_TOPK):
            hi = jnp.maximum(t[j], v)
            v = jnp.minimum(t[j], v)
            t[j] = hi
        return tuple(t)

    t0 = tuple(
        jnp.full((s, nb), -jnp.inf, dtype=jnp.float32) for _ in range(_TOPK)
    )
    return jax.lax.fori_loop(0, nchunks, _insert, t0, unroll=unroll)


def _extract_threshold(t, nb):
    vals = jnp.concatenate(list(t), axis=0)
    need = jnp.full((1, nb), float(_TOPK), dtype=jnp.float32)
    th = jnp.full((1, nb), -jnp.inf, dtype=jnp.float32)
    for _ in range(_TOPK):
        m = jnp.max(vals, axis=0, keepdims=True)
        ge = vals >= m
        c = jnp.sum(ge.astype(jnp.float32), axis=0, keepdims=True)
        th = jnp.where(need > 0.0, m, th)
        need = need - c
        vals = jnp.where(ge, -jnp.inf, vals)
    return th


def _assemble_body(ncols, aff_ref, feat_ref, out_ref, cand_ref):
    p = aff_ref.shape[1]
    nb = aff_ref.shape[2]
    full_groups = p // (_S * _G)
    tail_chunks = (p - full_groups * _S * _G) // _S

    def _prefilter(g, carry):
        base = g * _S * _G
        vs = [aff_ref[0, pl.ds(base + j * _S, _S), :] for j in range(8)]
        t0 = _top3_of8(vs)
        vs = [aff_ref[0, pl.ds(base + (8 + j) * _S, _S), :] for j in range(8)]
        t1 = _top3_of8(vs)
        top = _merge_sorted(t0, t1, _KEEP)
        for j in range(_KEEP):
            cand_ref[pl.ds((g * _KEEP + j) * _S, _S), :] = top[j]
        return carry

    jax.lax.fori_loop(0, full_groups, _prefilter, 0, unroll=2)
    if tail_chunks:
        base = full_groups * _S * _G
        vs = [
            aff_ref[0, pl.ds(base + j * _S, _S), :] for j in range(tail_chunks)
        ]
        top = _top3_of8(vs)
        for j in range(_KEEP):
            cand_ref[pl.ds(((full_groups * _KEEP) + j) * _S, _S), :] = top[j]

    ncand = (full_groups + 1) * _KEEP

    t = _insertion_topk(
        lambda i: cand_ref[pl.ds(i * _S, _S), :], ncand, _S, nb, 8
    )
    th1 = _extract_threshold(t, nb)
    mx = jnp.max(t[0], axis=0, keepdims=True)

    a = aff_ref[0]
    cnt_gt = jnp.sum((a > th1).astype(jnp.float32), axis=0, keepdims=True)
    base_col = pl.program_id(1) * nb
    colid = jax.lax.broadcasted_iota(jnp.int32, (1, nb), 1) + base_col
    valid = colid < ncols
    ok = jnp.all(jnp.where(valid, cnt_gt, 0.0) < float(_TOPK))

    def _fallback():
        tf = _insertion_topk(
            lambda i: aff_ref[0, pl.ds(i * _S, _S), :], p // _S, _S, nb, 8
        )
        return _extract_threshold(tf, nb)

    th = jax.lax.cond(ok, lambda: th1, _fallback)

    e = jnp.where(a >= th, jnp.exp(a - mx), 0.0)
    ssum = jnp.sum(e, axis=0, keepdims=True)
    f = feat_ref[0]
    acc = jax.lax.dot_general(
        f, e, (((1,), (0,)), ((), ())), preferred_element_type=jnp.float32
    )
    out_ref[0] = acc * (1.0 / ssum)


def kernel(cur_prev_aff, feat):
    import functools

    B, P, N = cur_prev_aff.shape
    C = feat.shape[1]
    NB = _NB
    grid = (B, pl.cdiv(N, NB))
    ncand_chunks = (P // (_S * _G) + 1) * _KEEP
    return pl.pallas_call(
        functools.partial(_assemble_body, N),
        grid=grid,
        in_specs=[
            pl.BlockSpec((1, P, NB), lambda b, n: (b, 0, n)),
            pl.BlockSpec((1, C, P), lambda b, n: (b, 0, 0)),
        ],
        out_specs=pl.BlockSpec((1, C, NB), lambda b, n: (b, 0, n)),
        out_shape=jax.ShapeDtypeStruct((B, C, N), jnp.float32),
        scratch_shapes=[pltpu.VMEM((ncand_chunks * _S, NB), jnp.float32)],
    )(cur_prev_aff, feat)
